# Initial kernel scaffold; baseline (speedup 1.0000x reference)
import jax
import jax.numpy as jnp
from jax.experimental import pallas as pl


def kernel(X, A, W_eq, b_eq, W_inv, b_inv):
    raise NotImplementedError("write your pallas kernel here")


# speedup vs baseline: 2.6591x; 2.6591x over previous
import jax
import jax.numpy as jnp
from jax import lax
from jax.experimental import pallas as pl

_B, _N, _F, _M, _H = 4, 4096, 64, 64, 8
_MH = _M * _H
_TR = 1024


def _phase1_body(a_ref, stats_ref):
    r = pl.program_id(1)
    a = a_ref[0]
    rowsum = jnp.sum(a, axis=1)
    csum = jnp.sum(a, axis=0)

    dblk = a_ref[0, :, pl.ds(r * _TR, _TR)]
    ii = lax.broadcasted_iota(jnp.int32, (_TR, _TR), 0)
    jj = lax.broadcasted_iota(jnp.int32, (_TR, _TR), 1)
    dg = jnp.sum(jnp.where(ii == jj, dblk, 0.0), axis=1)

    @pl.when(r == 0)
    def _():
        stats_ref[0] = jnp.zeros((_N, 8), jnp.float32)

    stats_ref[0, :, 0:1] += csum[:, None]
    stats_ref[0, pl.ds(r * _TR, _TR), 1:2] = rowsum[:, None]
    stats_ref[0, pl.ds(r * _TR, _TR), 2:3] = dg[:, None]


def _phase2_body(x_ref, stats_ref, wn_ref, w2n_ref, wa_ref, sel_ref, binv_ref,
                 psi_ref):
    inv_n = 1.0 / _N
    x = x_ref[0]
    h1 = jnp.dot(x, wn_ref[...], preferred_element_type=jnp.float32)
    sumx = jnp.sum(x, axis=0, keepdims=True)
    base = jnp.dot(sumx, w2n_ref[...], preferred_element_type=jnp.float32)
    cl = stats_ref[0, :, 0:1]
    rw = stats_ref[0, :, 1:2]
    dg = stats_ref[0, :, 2:3]
    sum_diag = jnp.sum(dg)
    suma = jnp.sum(rw)
    wa = wa_ref[...]
    base = (base + (sum_diag * inv_n) * wa[1:2]
            + (suma * inv_n * inv_n) * wa[4:5] + wa[5:6])
    pernode = (dg * wa[0:1] + (rw * inv_n) * wa[2:3]
               + (cl * inv_n) * wa[3:4])
    hidden = jnp.maximum(h1 + pernode + base, 0.0)
    s = jnp.sum(hidden, axis=0, keepdims=True)
    psi = jnp.dot(s, sel_ref[...], preferred_element_type=jnp.float32) * inv_n
    psi_ref[0, 0, :] = psi[0] + binv_ref[0]


def kernel(X, A, W_eq, b_eq, W_inv, b_inv):
    n = float(_N)
    Wx = W_eq[:, :, :_F, :]
    wav = W_eq[:, :, _F, :]
    Wn = (Wx[:, 0] + (Wx[:, 2] + Wx[:, 3]) * (1.0 / n))
    Wn = jnp.transpose(Wn, (1, 0, 2)).reshape(_F, _MH)
    W2n = (Wx[:, 1] * (1.0 / n) + Wx[:, 4] * (1.0 / (n * n)))
    W2n = jnp.transpose(W2n, (1, 0, 2)).reshape(_F, _MH)
    wa_rows = [wav[:, p].reshape(_MH) for p in range(5)]
    wa_pack = jnp.stack(wa_rows + [b_eq.reshape(_MH),
                                   jnp.zeros((_MH,), jnp.float32),
                                   jnp.zeros((_MH,), jnp.float32)])
    mh_ids = jnp.arange(_MH, dtype=jnp.int32) // _H
    sel = jnp.where(mh_ids[:, None] == jnp.arange(_M, dtype=jnp.int32)[None, :],
                    W_inv.reshape(_MH)[:, None], 0.0)

    R = _N // _TR
    stats = pl.pallas_call(
        _phase1_body,
        grid=(_B, R),
        in_specs=[pl.BlockSpec((1, _TR, _N), lambda b, r: (b, r, 0))],
        out_specs=pl.BlockSpec((1, _N, 8), lambda b, r: (b, 0, 0)),
        out_shape=jax.ShapeDtypeStruct((_B, _N, 8), jnp.float32),
    )(A)

    psi = pl.pallas_call(
        _phase2_body,
        grid=(_B,),
        in_specs=[
            pl.BlockSpec((1, _N, _F), lambda b: (b, 0, 0)),
            pl.BlockSpec((1, _N, 8), lambda b: (b, 0, 0)),
            pl.BlockSpec((_F, _MH), lambda b: (0, 0)),
            pl.BlockSpec((_F, _MH), lambda b: (0, 0)),
            pl.BlockSpec((8, _MH), lambda b: (0, 0)),
            pl.BlockSpec((_MH, _M), lambda b: (0, 0)),
            pl.BlockSpec((1, _M), lambda b: (0, 0)),
        ],
        out_specs=pl.BlockSpec((1, 1, _M), lambda b: (b, 0, 0)),
        out_shape=jax.ShapeDtypeStruct((_B, 1, _M), jnp.float32),
    )(X, stats, Wn, W2n, wa_pack, sel, b_inv.reshape(1, _M))
    return psi.reshape(_B, _M)
